# Initial kernel scaffold; baseline (speedup 1.0000x reference)
import jax
import jax.numpy as jnp
from jax.experimental import pallas as pl


def kernel(x, edge_index, batch, global_feat, W1l, W1r, att1, b1, g1, be1, W2l, W2r, att2, b2, g2, be2, fcW1, fcb1, fcW2, fcb2):
    raise NotImplementedError("write your pallas kernel here")


# speedup vs baseline: 7.6734x; 7.6734x over previous
import functools

import jax
import jax.numpy as jnp
from jax import lax
from jax.experimental import pallas as pl
from jax.experimental.pallas import tpu as pltpu
from jax.experimental.pallas import tpu_sc as plsc

N = 10000
E = 320000
G = 256
GF = 179
H = 4

NB = 25
BR = N // NB

NW = 32
CHUNK = 2000
GRP = CHUNK // 16
NCH = E // CHUNK
GB = 32
STG = 2112


def _pre_body(x_ref, wl_ref, wr_ref, xl_ref, xr_ref):
    xb = x_ref[...]
    xl_ref[...] = jnp.dot(xb, wl_ref[...], preferred_element_type=jnp.float32)
    xr_ref[...] = jnp.dot(xb, wr_ref[...], preferred_element_type=jnp.float32)


def _tc_two_matmuls(x, Wl, Wr):
    f = x.shape[1]
    ct = Wl.shape[1]
    return pl.pallas_call(
        _pre_body,
        grid=(NB,),
        in_specs=[
            pl.BlockSpec((BR, f), lambda i: (i, 0)),
            pl.BlockSpec((f, ct), lambda i: (0, 0)),
            pl.BlockSpec((f, ct), lambda i: (0, 0)),
        ],
        out_specs=[pl.BlockSpec((BR, ct), lambda i: (i, 0)),
                   pl.BlockSpec((BR, ct), lambda i: (i, 0))],
        out_shape=[jax.ShapeDtypeStruct((N, ct), jnp.float32),
                   jax.ShapeDtypeStruct((N, ct), jnp.float32)],
    )(x, Wl, Wr)


def _stats_body(o_ref, b_ref, s_ref, q_ref):
    @pl.when(pl.program_id(0) == 0)
    def _():
        s_ref[...] = jnp.zeros_like(s_ref)
        q_ref[...] = jnp.zeros_like(q_ref)

    t = o_ref[...] + b_ref[...]
    s_ref[...] += jnp.sum(t, axis=0, keepdims=True)
    q_ref[...] += jnp.sum(t * t, axis=0, keepdims=True)


def _tc_stats(o, b):
    ct = o.shape[1]
    return pl.pallas_call(
        _stats_body,
        grid=(NB,),
        in_specs=[
            pl.BlockSpec((BR, ct), lambda i: (i, 0)),
            pl.BlockSpec((1, ct), lambda i: (0, 0)),
        ],
        out_specs=[pl.BlockSpec((1, ct), lambda i: (0, 0)),
                   pl.BlockSpec((1, ct), lambda i: (0, 0))],
        out_shape=[jax.ShapeDtypeStruct((1, ct), jnp.float32),
                   jax.ShapeDtypeStruct((1, ct), jnp.float32)],
    )(o, b)


def _mid_body(o_ref, b_ref, s_ref, q_ref, g_ref, be_ref, wl_ref, wr_ref,
              xl_ref, xr_ref):
    mu = s_ref[...] / N
    var = q_ref[...] / N - mu * mu
    rstd = lax.rsqrt(var + 1e-5)
    h = (o_ref[...] + b_ref[...] - mu) * rstd * g_ref[...] + be_ref[...]
    h = jnp.maximum(h, 0.0)
    xl_ref[...] = jnp.dot(h, wl_ref[...], preferred_element_type=jnp.float32)
    xr_ref[...] = jnp.dot(h, wr_ref[...], preferred_element_type=jnp.float32)


def _tc_mid(o, b, s, q, g, be, Wl, Wr):
    ct = o.shape[1]
    c2 = Wl.shape[1]
    vec = lambda i: (0, 0)
    return pl.pallas_call(
        _mid_body,
        grid=(NB,),
        in_specs=[
            pl.BlockSpec((BR, ct), lambda i: (i, 0)),
            pl.BlockSpec((1, ct), vec),
            pl.BlockSpec((1, ct), vec),
            pl.BlockSpec((1, ct), vec),
            pl.BlockSpec((1, ct), vec),
            pl.BlockSpec((1, ct), vec),
            pl.BlockSpec((ct, c2), lambda i: (0, 0)),
            pl.BlockSpec((ct, c2), lambda i: (0, 0)),
        ],
        out_specs=[pl.BlockSpec((BR, c2), lambda i: (i, 0)),
                   pl.BlockSpec((BR, c2), lambda i: (i, 0))],
        out_shape=[jax.ShapeDtypeStruct((N, c2), jnp.float32),
                   jax.ShapeDtypeStruct((N, c2), jnp.float32)],
    )(o, b, s, q, g, be, Wl, Wr)


def _post_body(o_ref, b_ref, s_ref, q_ref, g_ref, be_ref, batch_ref, ps_ref):
    @pl.when(pl.program_id(0) == 0)
    def _():
        ps_ref[...] = jnp.zeros_like(ps_ref)

    mu = s_ref[...] / N
    var = q_ref[...] / N - mu * mu
    rstd = lax.rsqrt(var + 1e-5)
    h = (o_ref[...] + b_ref[...] - mu) * rstd * g_ref[...] + be_ref[...]
    h = jnp.maximum(h, 0.0)
    hx = jnp.concatenate(
        [h, jnp.ones((BR, 128), jnp.float32)], axis=1)
    bvec = batch_ref[...].reshape(1, BR)
    onehot = (bvec == lax.broadcasted_iota(jnp.int32, (G, BR), 0))
    ps_ref[...] += jnp.dot(onehot.astype(jnp.float32), hx,
                           preferred_element_type=jnp.float32)


def _tc_post(o, b, s, q, g, be, batch3):
    ct = o.shape[1]
    vec = lambda i: (0, 0)
    return pl.pallas_call(
        _post_body,
        grid=(NB,),
        in_specs=[
            pl.BlockSpec((BR, ct), lambda i: (i, 0)),
            pl.BlockSpec((1, ct), vec),
            pl.BlockSpec((1, ct), vec),
            pl.BlockSpec((1, ct), vec),
            pl.BlockSpec((1, ct), vec),
            pl.BlockSpec((1, ct), vec),
            pl.BlockSpec((1, 1, BR), lambda i: (i, 0, 0)),
        ],
        out_specs=pl.BlockSpec((G, ct + 128), lambda i: (0, 0)),
        out_shape=jax.ShapeDtypeStruct((G, ct + 128), jnp.float32),
    )(o, b, s, q, g, be, batch3)


def _mlp_body(ps_ref, gf_ref, w1a_ref, w1b_ref, b1_ref, w2_ref, b2_ref, o_ref):
    ps = ps_ref[...]
    pooled = ps[:, :512]
    cnt = ps[:, 512:513]
    pooled = pooled / jnp.maximum(cnt, 1.0)
    z = (jnp.dot(pooled, w1a_ref[...], preferred_element_type=jnp.float32)
         + jnp.dot(gf_ref[...], w1b_ref[...], preferred_element_type=jnp.float32)
         + b1_ref[...])
    z = jnp.maximum(z, 0.0)
    o_ref[...] = jnp.dot(z, w2_ref[...], preferred_element_type=jnp.float32) \
        + b2_ref[...]


def _tc_mlp(ps, gf, w1a, w1b, b1, w2, b2):
    return pl.pallas_call(
        _mlp_body,
        out_shape=jax.ShapeDtypeStruct((G, 1), jnp.float32),
    )(ps, gf, w1a, w1b, b1, w2, b2)


def _make_sc_edge(ct, dn, base0):
    cl = ct // 16
    clh = cl // H
    mesh = plsc.VectorSubcoreMesh(core_axis_name="c", subcore_axis_name="s")

    def body(xl_hbm, xr_hbm, src_hbm, dst_hbm, att_hbm, out_hbm,
             sbuf, dbuf, attv, sstage, dstage, shead, dhead,
             rowsl, rowsr, md, outv, sem, sem2):
        wid = lax.axis_index("s") * 2 + lax.axis_index("c")
        base = base0 + wid * dn
        iota = lax.iota(jnp.int32, 16)
        f0 = jnp.zeros((16,), jnp.float32)

        pltpu.sync_copy(att_hbm, attv)

        def init_md(r, c):
            plsc.store_scatter(md, [r * 8 + (iota & 7)],
                               jnp.where(iota < 4, -3.0e38, 0.0),
                               mask=iota < 8)
            return c
        lax.fori_loop(0, dn, init_md, 0)

        def init_out(r, c):
            rfull = jnp.full((16,), r, jnp.int32)

            def init_j(j, c2):
                plsc.store_scatter(outv, [rfull, j * 16 + iota], f0)
                return c2
            lax.fori_loop(0, cl, init_j, 0)
            return c
        lax.fori_loop(0, dn, init_out, 0)

        def init_stage(k, c):
            plsc.store_scatter(sstage, [k * 16 + iota],
                               jnp.zeros((16,), jnp.int32))
            plsc.store_scatter(dstage, [k * 16 + iota],
                               jnp.zeros((16,), jnp.int32))
            return c
        lax.fori_loop(0, STG // 16, init_stage, 0)

        def process_batch(boff, pcount):
            for t in range(GB // 16):
                sh = plsc.load_gather(sstage, [boff + t * 16 + iota])
                dh = plsc.load_gather(dstage, [boff + t * 16 + iota])
                shead[pl.ds(t * 16, 16)] = sh
                dhead[pl.ds(t * 16, 16)] = dh
            cpa = pltpu.async_copy(xl_hbm.at[shead], rowsl, sem)
            cpb = pltpu.async_copy(xr_hbm.at[dhead], rowsr, sem2)
            cpa.wait()
            cpb.wait()

            def edge_body(i, c):
                ifull = jnp.full((16,), i, jnp.int32)
                dspl = plsc.load_gather(dhead, [ifull])
                drel = dspl - base
                st = plsc.load_gather(md, [drel * 8 + (iota & 7)],
                                      mask=iota < 8)

                rowregs = []
                l4 = f0
                for h in range(H):
                    a1 = f0
                    a2 = f0
                    for j in range(clh):
                        jj = h * clh + j
                        col = jj * 16 + iota
                        vl = plsc.load_gather(rowsl, [ifull, col])
                        vr = plsc.load_gather(rowsr, [ifull, col])
                        av = attv[pl.ds(jj * 16, 16)]
                        v = vl + vr
                        a1 = a1 + av * v
                        a2 = a2 + av * jnp.maximum(v, 0.0)
                        rowregs.append(vl)
                    lh = jnp.sum(0.2 * a1 + 0.8 * a2)
                    l4 = jnp.where(iota == h, lh, l4)

                den_al = st.at[jnp.minimum(iota + 4, 15)].get(
                    mode="promise_in_bounds")
                nm = jnp.maximum(st, l4)
                scl = jnp.exp(st - nm)
                w = jnp.exp(l4 - nm)
                den_new = den_al * scl + w
                den_sh = den_new.at[jnp.maximum(iota - 4, 0)].get(
                    mode="promise_in_bounds")
                stn = jnp.where(iota < 4, nm, den_sh)
                plsc.store_scatter(md, [drel * 8 + (iota & 7)], stn,
                                   mask=iota < 8)

                for h in range(H):
                    wh = w.at[jnp.full((16,), h, jnp.int32)].get(
                        mode="promise_in_bounds")
                    sh_ = scl.at[jnp.full((16,), h, jnp.int32)].get(
                        mode="promise_in_bounds")
                    for j in range(clh):
                        jj = h * clh + j
                        col = jj * 16 + iota
                        acc = plsc.load_gather(outv, [drel, col])
                        acc = acc * sh_ + wh * rowregs[jj]
                        plsc.store_scatter(outv, [drel, col], acc)
                return c
            lax.fori_loop(0, pcount, edge_body, 0)

        def chunk_body(ci, cnt):
            pltpu.sync_copy(src_hbm.at[pl.ds(ci * CHUNK, CHUNK)], sbuf)
            pltpu.sync_copy(dst_hbm.at[pl.ds(ci * CHUNK, CHUNK)], dbuf)

            def grp_body(k, c):
                s = plsc.load_gather(sbuf, [k * 16 + iota])
                d = plsc.load_gather(dbuf, [k * 16 + iota])
                drel = d - base
                hit = (drel >= 0) & (drel < dn)
                hi = hit.astype(jnp.int32)
                nh = jnp.sum(hi)
                pos = jnp.maximum(c + plsc.cumsum(hi) - 1, 0)
                plsc.store_scatter(sstage, [pos], s, mask=hit)
                plsc.store_scatter(dstage, [pos], d, mask=hit)
                return c + nh
            cnt = lax.fori_loop(0, GRP, grp_body, cnt)

            nb = cnt // GB

            def batch_body(b, c):
                process_batch(b * GB, GB)
                return c
            lax.fori_loop(0, nb, batch_body, 0)

            rem = cnt - nb * GB
            roff = nb * GB
            for t in range(2):
                rs = plsc.load_gather(sstage, [roff + t * 16 + iota])
                rd = plsc.load_gather(dstage, [roff + t * 16 + iota])
                mk = (t * 16 + iota) < rem
                plsc.store_scatter(sstage, [t * 16 + iota], rs, mask=mk)
                plsc.store_scatter(dstage, [t * 16 + iota], rd, mask=mk)
            return rem
        cnt = lax.fori_loop(0, NCH, chunk_body, 0)

        @pl.when(cnt > 0)
        def _():
            process_batch(0, cnt)

        def norm_body(r, c):
            rfull = jnp.full((16,), r, jnp.int32)
            st = plsc.load_gather(md, [r * 8 + (iota & 7)], mask=iota < 8)
            for h in range(H):
                dh = st.at[jnp.full((16,), 4 + h, jnp.int32)].get(
                    mode="promise_in_bounds")
                inv = 1.0 / (dh + 1e-16)
                for j in range(clh):
                    col = (h * clh + j) * 16 + iota
                    v = plsc.load_gather(outv, [rfull, col])
                    plsc.store_scatter(outv, [rfull, col], v * inv)
            return c
        lax.fori_loop(0, dn, norm_body, 0)

        pltpu.sync_copy(outv, out_hbm.at[pl.ds(wid * dn, dn)])

    fn = functools.partial(
        pl.kernel, body,
        mesh=mesh,
        compiler_params=pltpu.CompilerParams(use_tc_tiling_on_sc=False,
                                             needs_layout_passes=False),
        out_type=jax.ShapeDtypeStruct((NW * dn, ct), jnp.float32),
        scratch_types=[
            pltpu.VMEM((CHUNK,), jnp.int32),
            pltpu.VMEM((CHUNK,), jnp.int32),
            pltpu.VMEM((ct,), jnp.float32),
            pltpu.VMEM((STG,), jnp.int32),
            pltpu.VMEM((STG,), jnp.int32),
            pltpu.VMEM((GB,), jnp.int32),
            pltpu.VMEM((GB,), jnp.int32),
            pltpu.VMEM((GB, ct), jnp.float32),
            pltpu.VMEM((GB, ct), jnp.float32),
            pltpu.VMEM((dn * 8,), jnp.float32),
            pltpu.VMEM((dn, ct), jnp.float32),
            pltpu.SemaphoreType.DMA,
            pltpu.SemaphoreType.DMA,
        ],
    )
    return fn()


def _gat_layer_sc(xl, xr, src, dst, att, splits):
    ct = xl.shape[1]
    att_flat = att.reshape(ct)
    outs = []
    for dn, base0 in splits:
        sc = _make_sc_edge(ct, dn, base0)
        outs.append(sc(xl, xr, src, dst, att_flat))
    o = outs[0] if len(outs) == 1 else jnp.concatenate(outs, axis=0)
    return o[:N]


def kernel(x, edge_index, batch, global_feat, W1l, W1r, att1, b1, g1, be1,
           W2l, W2r, att2, b2, g2, be2, fcW1, fcb1, fcW2, fcb2):
    src = edge_index[0]
    dst = edge_index[1]

    xl1, xr1 = _tc_two_matmuls(x, W1l, W1r)
    o1 = _gat_layer_sc(xl1, xr1, src, dst, att1, [(320, 0)])
    s1, q1 = _tc_stats(o1, b1.reshape(1, -1))
    xl2, xr2 = _tc_mid(o1, b1.reshape(1, -1), s1, q1, g1.reshape(1, -1),
                       be1.reshape(1, -1), W2l, W2r)

    o2 = _gat_layer_sc(xl2, xr2, src, dst, att2, [(160, 0), (160, 5120)])
    s2, q2 = _tc_stats(o2, b2.reshape(1, -1))

    batch3 = batch.reshape(NB, 1, BR)
    ps = _tc_post(o2, b2.reshape(1, -1), s2, q2, g2.reshape(1, -1),
                  be2.reshape(1, -1), batch3)

    out = _tc_mlp(ps, global_feat, fcW1[:512], fcW1[512:],
                  fcb1.reshape(1, -1), fcW2, fcb2.reshape(1, 1))
    return out.squeeze()


# speedup vs baseline: 12.4904x; 1.6278x over previous
import functools

import jax
import jax.numpy as jnp
from jax import lax
from jax.experimental import pallas as pl
from jax.experimental.pallas import tpu as pltpu
from jax.experimental.pallas import tpu_sc as plsc

N = 10000
E = 320000
G = 256
GF = 179
H = 4

NB = 25
BR = N // NB

NW = 32
CHUNK = 2000
GRP = CHUNK // 16
NCH = E // CHUNK
STG = 2112


def _pre_body(x_ref, wl_ref, wr_ref, xl_ref, xr_ref):
    xb = x_ref[...]
    xl_ref[...] = jnp.dot(xb, wl_ref[...], preferred_element_type=jnp.float32)
    xr_ref[...] = jnp.dot(xb, wr_ref[...], preferred_element_type=jnp.float32)


def _tc_two_matmuls(x, Wl, Wr):
    f = x.shape[1]
    ct = Wl.shape[1]
    return pl.pallas_call(
        _pre_body,
        grid=(NB,),
        in_specs=[
            pl.BlockSpec((BR, f), lambda i: (i, 0)),
            pl.BlockSpec((f, ct), lambda i: (0, 0)),
            pl.BlockSpec((f, ct), lambda i: (0, 0)),
        ],
        out_specs=[pl.BlockSpec((BR, ct), lambda i: (i, 0)),
                   pl.BlockSpec((BR, ct), lambda i: (i, 0))],
        out_shape=[jax.ShapeDtypeStruct((N, ct), jnp.float32),
                   jax.ShapeDtypeStruct((N, ct), jnp.float32)],
    )(x, Wl, Wr)


def _stats_body(o_ref, b_ref, s_ref, q_ref):
    @pl.when(pl.program_id(0) == 0)
    def _():
        s_ref[...] = jnp.zeros_like(s_ref)
        q_ref[...] = jnp.zeros_like(q_ref)

    t = o_ref[...] + b_ref[...]
    s_ref[...] += jnp.sum(t, axis=0, keepdims=True)
    q_ref[...] += jnp.sum(t * t, axis=0, keepdims=True)


def _tc_stats(o, b):
    ct = o.shape[1]
    return pl.pallas_call(
        _stats_body,
        grid=(NB,),
        in_specs=[
            pl.BlockSpec((BR, ct), lambda i: (i, 0)),
            pl.BlockSpec((1, ct), lambda i: (0, 0)),
        ],
        out_specs=[pl.BlockSpec((1, ct), lambda i: (0, 0)),
                   pl.BlockSpec((1, ct), lambda i: (0, 0))],
        out_shape=[jax.ShapeDtypeStruct((1, ct), jnp.float32),
                   jax.ShapeDtypeStruct((1, ct), jnp.float32)],
    )(o, b)


def _mid_body(o_ref, b_ref, s_ref, q_ref, g_ref, be_ref, wl_ref, wr_ref,
              xl_ref, xr_ref):
    mu = s_ref[...] / N
    var = q_ref[...] / N - mu * mu
    rstd = lax.rsqrt(var + 1e-5)
    h = (o_ref[...] + b_ref[...] - mu) * rstd * g_ref[...] + be_ref[...]
    h = jnp.maximum(h, 0.0)
    xl_ref[...] = jnp.dot(h, wl_ref[...], preferred_element_type=jnp.float32)
    xr_ref[...] = jnp.dot(h, wr_ref[...], preferred_element_type=jnp.float32)


def _tc_mid(o, b, s, q, g, be, Wl, Wr):
    ct = o.shape[1]
    c2 = Wl.shape[1]
    vec = lambda i: (0, 0)
    return pl.pallas_call(
        _mid_body,
        grid=(NB,),
        in_specs=[
            pl.BlockSpec((BR, ct), lambda i: (i, 0)),
            pl.BlockSpec((1, ct), vec),
            pl.BlockSpec((1, ct), vec),
            pl.BlockSpec((1, ct), vec),
            pl.BlockSpec((1, ct), vec),
            pl.BlockSpec((1, ct), vec),
            pl.BlockSpec((ct, c2), lambda i: (0, 0)),
            pl.BlockSpec((ct, c2), lambda i: (0, 0)),
        ],
        out_specs=[pl.BlockSpec((BR, c2), lambda i: (i, 0)),
                   pl.BlockSpec((BR, c2), lambda i: (i, 0))],
        out_shape=[jax.ShapeDtypeStruct((N, c2), jnp.float32),
                   jax.ShapeDtypeStruct((N, c2), jnp.float32)],
    )(o, b, s, q, g, be, Wl, Wr)


def _post_body(o_ref, b_ref, s_ref, q_ref, g_ref, be_ref, batch_ref, ps_ref):
    @pl.when(pl.program_id(0) == 0)
    def _():
        ps_ref[...] = jnp.zeros_like(ps_ref)

    mu = s_ref[...] / N
    var = q_ref[...] / N - mu * mu
    rstd = lax.rsqrt(var + 1e-5)
    h = (o_ref[...] + b_ref[...] - mu) * rstd * g_ref[...] + be_ref[...]
    h = jnp.maximum(h, 0.0)
    hx = jnp.concatenate(
        [h, jnp.ones((BR, 128), jnp.float32)], axis=1)
    bvec = batch_ref[...].reshape(1, BR)
    onehot = (bvec == lax.broadcasted_iota(jnp.int32, (G, BR), 0))
    ps_ref[...] += jnp.dot(onehot.astype(jnp.float32), hx,
                           preferred_element_type=jnp.float32)


def _tc_post(o, b, s, q, g, be, batch3):
    ct = o.shape[1]
    vec = lambda i: (0, 0)
    return pl.pallas_call(
        _post_body,
        grid=(NB,),
        in_specs=[
            pl.BlockSpec((BR, ct), lambda i: (i, 0)),
            pl.BlockSpec((1, ct), vec),
            pl.BlockSpec((1, ct), vec),
            pl.BlockSpec((1, ct), vec),
            pl.BlockSpec((1, ct), vec),
            pl.BlockSpec((1, ct), vec),
            pl.BlockSpec((1, 1, BR), lambda i: (i, 0, 0)),
        ],
        out_specs=pl.BlockSpec((G, ct + 128), lambda i: (0, 0)),
        out_shape=jax.ShapeDtypeStruct((G, ct + 128), jnp.float32),
    )(o, b, s, q, g, be, batch3)


def _mlp_body(ps_ref, gf_ref, w1a_ref, w1b_ref, b1_ref, w2_ref, b2_ref, o_ref):
    ps = ps_ref[...]
    pooled = ps[:, :512]
    cnt = ps[:, 512:513]
    pooled = pooled / jnp.maximum(cnt, 1.0)
    z = (jnp.dot(pooled, w1a_ref[...], preferred_element_type=jnp.float32)
         + jnp.dot(gf_ref[...], w1b_ref[...], preferred_element_type=jnp.float32)
         + b1_ref[...])
    z = jnp.maximum(z, 0.0)
    o_ref[...] = jnp.dot(z, w2_ref[...], preferred_element_type=jnp.float32) \
        + b2_ref[...]


def _tc_mlp(ps, gf, w1a, w1b, b1, w2, b2):
    return pl.pallas_call(
        _mlp_body,
        out_shape=jax.ShapeDtypeStruct((G, 1), jnp.float32),
    )(ps, gf, w1a, w1b, b1, w2, b2)


def _make_sc_edge(ct, dn, base0, GB):
    cl = ct // 16
    clh = cl // H
    mesh = plsc.VectorSubcoreMesh(core_axis_name="c", subcore_axis_name="s")

    def body(xl_hbm, xr_hbm, src_hbm, dst_hbm, att_hbm, out_hbm,
             sbuf, dbuf, attv, sstage, dstage, shead, dhead,
             rowsl, rowsr, md, outv, sem, sem2):
        wid = lax.axis_index("s") * 2 + lax.axis_index("c")
        base = base0 + wid * dn
        iota = lax.iota(jnp.int32, 16)
        f0 = jnp.zeros((16,), jnp.float32)

        pltpu.sync_copy(att_hbm, attv)

        def init_md(r, c):
            plsc.store_scatter(md, [r * 8 + (iota & 7)],
                               jnp.where(iota < 4, -3.0e38, 0.0),
                               mask=iota < 8)
            return c
        lax.fori_loop(0, dn, init_md, 0)

        def init_out(r, c):
            def init_j(j, c2):
                outv[r, pl.ds(j * 16, 16)] = f0
                return c2
            lax.fori_loop(0, cl, init_j, 0)
            return c
        lax.fori_loop(0, dn, init_out, 0)

        def init_stage(k, c):
            plsc.store_scatter(sstage, [k * 16 + iota],
                               jnp.zeros((16,), jnp.int32))
            plsc.store_scatter(dstage, [k * 16 + iota],
                               jnp.zeros((16,), jnp.int32))
            return c
        lax.fori_loop(0, STG // 16, init_stage, 0)

        def process_batch(boff, pcount):
            for t in range(GB // 16):
                sh = plsc.load_gather(sstage, [boff + t * 16 + iota])
                dh = plsc.load_gather(dstage, [boff + t * 16 + iota])
                shead[pl.ds(t * 16, 16)] = sh
                dhead[pl.ds(t * 16, 16)] = dh
            cpa = pltpu.async_copy(xl_hbm.at[shead], rowsl, sem)
            cpb = pltpu.async_copy(xr_hbm.at[dhead], rowsr, sem2)
            cpa.wait()
            cpb.wait()

            def edge_body(i, c):
                ifull = jnp.full((16,), i, jnp.int32)
                dspl = plsc.load_gather(dhead, [ifull])
                drel = dspl - base
                dsc = jnp.max(drel)
                st = md[pl.ds(dsc * 8, 16)]

                rowregs = []
                l4 = f0
                for h in range(H):
                    a1 = f0
                    a2 = f0
                    for j in range(clh):
                        jj = h * clh + j
                        col = jj * 16 + iota
                        vl = rowsl[i, pl.ds(jj * 16, 16)]
                        vr = rowsr[i, pl.ds(jj * 16, 16)]
                        av = attv[pl.ds(jj * 16, 16)]
                        v = vl + vr
                        a1 = a1 + av * v
                        a2 = a2 + av * jnp.maximum(v, 0.0)
                        rowregs.append(vl)
                    lh = jnp.sum(0.2 * a1 + 0.8 * a2)
                    l4 = jnp.where(iota == h, lh, l4)

                den_al = st.at[jnp.minimum(iota + 4, 15)].get(
                    mode="promise_in_bounds")
                nm = jnp.maximum(st, l4)
                scl = jnp.exp(st - nm)
                w = jnp.exp(l4 - nm)
                den_new = den_al * scl + w
                den_sh = den_new.at[jnp.maximum(iota - 4, 0)].get(
                    mode="promise_in_bounds")
                stn = jnp.where(iota < 4, nm, den_sh)
                plsc.store_scatter(md, [dsc * 8 + (iota & 7)], stn,
                                   mask=iota < 8)

                for h in range(H):
                    wh = w.at[jnp.full((16,), h, jnp.int32)].get(
                        mode="promise_in_bounds")
                    sh_ = scl.at[jnp.full((16,), h, jnp.int32)].get(
                        mode="promise_in_bounds")
                    for j in range(clh):
                        jj = h * clh + j
                        col = jj * 16 + iota
                        acc = outv[dsc, pl.ds(jj * 16, 16)]
                        acc = acc * sh_ + wh * rowregs[jj]
                        outv[dsc, pl.ds(jj * 16, 16)] = acc
                return c
            lax.fori_loop(0, pcount, edge_body, 0)

        def chunk_body(ci, cnt):
            pltpu.sync_copy(src_hbm.at[pl.ds(ci * CHUNK, CHUNK)], sbuf)
            pltpu.sync_copy(dst_hbm.at[pl.ds(ci * CHUNK, CHUNK)], dbuf)

            def grp_body(k, c):
                s = plsc.load_gather(sbuf, [k * 16 + iota])
                d = plsc.load_gather(dbuf, [k * 16 + iota])
                drel = d - base
                hit = (drel >= 0) & (drel < dn)
                hi = hit.astype(jnp.int32)
                nh = jnp.sum(hi)
                pos = jnp.maximum(c + plsc.cumsum(hi) - 1, 0)
                plsc.store_scatter(sstage, [pos], s, mask=hit)
                plsc.store_scatter(dstage, [pos], d, mask=hit)
                return c + nh
            cnt = lax.fori_loop(0, GRP, grp_body, cnt)

            nb = cnt // GB

            def batch_body(b, c):
                process_batch(b * GB, GB)
                return c
            lax.fori_loop(0, nb, batch_body, 0)

            rem = cnt - nb * GB
            roff = nb * GB
            for t in range(GB // 16):
                rs = plsc.load_gather(sstage, [roff + t * 16 + iota])
                rd = plsc.load_gather(dstage, [roff + t * 16 + iota])
                mk = (t * 16 + iota) < rem
                plsc.store_scatter(sstage, [t * 16 + iota], rs, mask=mk)
                plsc.store_scatter(dstage, [t * 16 + iota], rd, mask=mk)
            return rem
        cnt = lax.fori_loop(0, NCH, chunk_body, 0)

        @pl.when(cnt > 0)
        def _():
            process_batch(0, cnt)

        def norm_body(r, c):
            st = md[pl.ds(r * 8, 16)]
            for h in range(H):
                dh = st.at[jnp.full((16,), 4 + h, jnp.int32)].get(
                    mode="promise_in_bounds")
                inv = 1.0 / (dh + 1e-16)
                for j in range(clh):
                    jj = h * clh + j
                    v = outv[r, pl.ds(jj * 16, 16)]
                    outv[r, pl.ds(jj * 16, 16)] = v * inv
            return c
        lax.fori_loop(0, dn, norm_body, 0)

        pltpu.sync_copy(outv, out_hbm.at[pl.ds(wid * dn, dn)])

    fn = functools.partial(
        pl.kernel, body,
        mesh=mesh,
        compiler_params=pltpu.CompilerParams(use_tc_tiling_on_sc=False,
                                             needs_layout_passes=False),
        out_type=jax.ShapeDtypeStruct((NW * dn, ct), jnp.float32),
        scratch_types=[
            pltpu.VMEM((CHUNK,), jnp.int32),
            pltpu.VMEM((CHUNK,), jnp.int32),
            pltpu.VMEM((ct,), jnp.float32),
            pltpu.VMEM((STG,), jnp.int32),
            pltpu.VMEM((STG,), jnp.int32),
            pltpu.VMEM((GB,), jnp.int32),
            pltpu.VMEM((GB,), jnp.int32),
            pltpu.VMEM((GB, ct), jnp.float32),
            pltpu.VMEM((GB, ct), jnp.float32),
            pltpu.VMEM((dn * 8 + 8,), jnp.float32),
            pltpu.VMEM((dn, ct), jnp.float32),
            pltpu.SemaphoreType.DMA,
            pltpu.SemaphoreType.DMA,
        ],
    )
    return fn()


def _gat_layer_sc(xl, xr, src, dst, att, splits):
    ct = xl.shape[1]
    att_flat = att.reshape(ct)
    outs = []
    gb = 64 if ct <= 256 else 32
    for dn, base0 in splits:
        sc = _make_sc_edge(ct, dn, base0, gb)
        outs.append(sc(xl, xr, src, dst, att_flat))
    o = outs[0] if len(outs) == 1 else jnp.concatenate(outs, axis=0)
    return o[:N]


def kernel(x, edge_index, batch, global_feat, W1l, W1r, att1, b1, g1, be1,
           W2l, W2r, att2, b2, g2, be2, fcW1, fcb1, fcW2, fcb2):
    src = edge_index[0]
    dst = edge_index[1]

    xl1, xr1 = _tc_two_matmuls(x, W1l, W1r)
    o1 = _gat_layer_sc(xl1, xr1, src, dst, att1, [(320, 0)])
    s1, q1 = _tc_stats(o1, b1.reshape(1, -1))
    xl2, xr2 = _tc_mid(o1, b1.reshape(1, -1), s1, q1, g1.reshape(1, -1),
                       be1.reshape(1, -1), W2l, W2r)

    o2 = _gat_layer_sc(xl2, xr2, src, dst, att2, [(160, 0), (160, 5120)])
    s2, q2 = _tc_stats(o2, b2.reshape(1, -1))

    batch3 = batch.reshape(NB, 1, BR)
    ps = _tc_post(o2, b2.reshape(1, -1), s2, q2, g2.reshape(1, -1),
                  be2.reshape(1, -1), batch3)

    out = _tc_mlp(ps, global_feat, fcW1[:512], fcW1[512:],
                  fcb1.reshape(1, -1), fcW2, fcb2.reshape(1, 1))
    return out.squeeze()


# speedup vs baseline: 14.5691x; 1.1664x over previous
import functools

import jax
import jax.numpy as jnp
from jax import lax
from jax.experimental import pallas as pl
from jax.experimental.pallas import tpu as pltpu
from jax.experimental.pallas import tpu_sc as plsc

N = 10000
E = 320000
G = 256
GF = 179
H = 4

NB = 25
BR = N // NB

NW = 32
CHUNK = 2000
GRP = CHUNK // 16
NCH = E // CHUNK
STG = 2112


def _pre_body(x_ref, wl_ref, wr_ref, xl_ref, xr_ref):
    xb = x_ref[...]
    xl_ref[...] = jnp.dot(xb, wl_ref[...], preferred_element_type=jnp.float32)
    xr_ref[...] = jnp.dot(xb, wr_ref[...], preferred_element_type=jnp.float32)


def _tc_two_matmuls(x, Wl, Wr):
    f = x.shape[1]
    ct = Wl.shape[1]
    return pl.pallas_call(
        _pre_body,
        grid=(NB,),
        in_specs=[
            pl.BlockSpec((BR, f), lambda i: (i, 0)),
            pl.BlockSpec((f, ct), lambda i: (0, 0)),
            pl.BlockSpec((f, ct), lambda i: (0, 0)),
        ],
        out_specs=[pl.BlockSpec((BR, ct), lambda i: (i, 0)),
                   pl.BlockSpec((BR, ct), lambda i: (i, 0))],
        out_shape=[jax.ShapeDtypeStruct((N, ct), jnp.float32),
                   jax.ShapeDtypeStruct((N, ct), jnp.float32)],
    )(x, Wl, Wr)


def _stats_body(o_ref, b_ref, s_ref, q_ref):
    @pl.when(pl.program_id(0) == 0)
    def _():
        s_ref[...] = jnp.zeros_like(s_ref)
        q_ref[...] = jnp.zeros_like(q_ref)

    t = o_ref[...] + b_ref[...]
    s_ref[...] += jnp.sum(t, axis=0, keepdims=True)
    q_ref[...] += jnp.sum(t * t, axis=0, keepdims=True)


def _tc_stats(o, b):
    ct = o.shape[1]
    return pl.pallas_call(
        _stats_body,
        grid=(NB,),
        in_specs=[
            pl.BlockSpec((BR, ct), lambda i: (i, 0)),
            pl.BlockSpec((1, ct), lambda i: (0, 0)),
        ],
        out_specs=[pl.BlockSpec((1, ct), lambda i: (0, 0)),
                   pl.BlockSpec((1, ct), lambda i: (0, 0))],
        out_shape=[jax.ShapeDtypeStruct((1, ct), jnp.float32),
                   jax.ShapeDtypeStruct((1, ct), jnp.float32)],
    )(o, b)


def _mid_body(o_ref, b_ref, s_ref, q_ref, g_ref, be_ref, wl_ref, wr_ref,
              xl_ref, xr_ref):
    mu = s_ref[...] / N
    var = q_ref[...] / N - mu * mu
    rstd = lax.rsqrt(var + 1e-5)
    h = (o_ref[...] + b_ref[...] - mu) * rstd * g_ref[...] + be_ref[...]
    h = jnp.maximum(h, 0.0)
    xl_ref[...] = jnp.dot(h, wl_ref[...], preferred_element_type=jnp.float32)
    xr_ref[...] = jnp.dot(h, wr_ref[...], preferred_element_type=jnp.float32)


def _tc_mid(o, b, s, q, g, be, Wl, Wr):
    ct = o.shape[1]
    c2 = Wl.shape[1]
    vec = lambda i: (0, 0)
    return pl.pallas_call(
        _mid_body,
        grid=(NB,),
        in_specs=[
            pl.BlockSpec((BR, ct), lambda i: (i, 0)),
            pl.BlockSpec((1, ct), vec),
            pl.BlockSpec((1, ct), vec),
            pl.BlockSpec((1, ct), vec),
            pl.BlockSpec((1, ct), vec),
            pl.BlockSpec((1, ct), vec),
            pl.BlockSpec((ct, c2), lambda i: (0, 0)),
            pl.BlockSpec((ct, c2), lambda i: (0, 0)),
        ],
        out_specs=[pl.BlockSpec((BR, c2), lambda i: (i, 0)),
                   pl.BlockSpec((BR, c2), lambda i: (i, 0))],
        out_shape=[jax.ShapeDtypeStruct((N, c2), jnp.float32),
                   jax.ShapeDtypeStruct((N, c2), jnp.float32)],
    )(o, b, s, q, g, be, Wl, Wr)


def _post_body(o_ref, b_ref, s_ref, q_ref, g_ref, be_ref, batch_ref, ps_ref):
    @pl.when(pl.program_id(0) == 0)
    def _():
        ps_ref[...] = jnp.zeros_like(ps_ref)

    mu = s_ref[...] / N
    var = q_ref[...] / N - mu * mu
    rstd = lax.rsqrt(var + 1e-5)
    h = (o_ref[...] + b_ref[...] - mu) * rstd * g_ref[...] + be_ref[...]
    h = jnp.maximum(h, 0.0)
    hx = jnp.concatenate(
        [h, jnp.ones((BR, 128), jnp.float32)], axis=1)
    bvec = batch_ref[...].reshape(1, BR)
    onehot = (bvec == lax.broadcasted_iota(jnp.int32, (G, BR), 0))
    ps_ref[...] += jnp.dot(onehot.astype(jnp.float32), hx,
                           preferred_element_type=jnp.float32)


def _tc_post(o, b, s, q, g, be, batch3):
    ct = o.shape[1]
    vec = lambda i: (0, 0)
    return pl.pallas_call(
        _post_body,
        grid=(NB,),
        in_specs=[
            pl.BlockSpec((BR, ct), lambda i: (i, 0)),
            pl.BlockSpec((1, ct), vec),
            pl.BlockSpec((1, ct), vec),
            pl.BlockSpec((1, ct), vec),
            pl.BlockSpec((1, ct), vec),
            pl.BlockSpec((1, ct), vec),
            pl.BlockSpec((1, 1, BR), lambda i: (i, 0, 0)),
        ],
        out_specs=pl.BlockSpec((G, ct + 128), lambda i: (0, 0)),
        out_shape=jax.ShapeDtypeStruct((G, ct + 128), jnp.float32),
    )(o, b, s, q, g, be, batch3)


def _mlp_body(ps_ref, gf_ref, w1a_ref, w1b_ref, b1_ref, w2_ref, b2_ref, o_ref):
    ps = ps_ref[...]
    pooled = ps[:, :512]
    cnt = ps[:, 512:513]
    pooled = pooled / jnp.maximum(cnt, 1.0)
    z = (jnp.dot(pooled, w1a_ref[...], preferred_element_type=jnp.float32)
         + jnp.dot(gf_ref[...], w1b_ref[...], preferred_element_type=jnp.float32)
         + b1_ref[...])
    z = jnp.maximum(z, 0.0)
    o_ref[...] = jnp.dot(z, w2_ref[...], preferred_element_type=jnp.float32) \
        + b2_ref[...]


def _tc_mlp(ps, gf, w1a, w1b, b1, w2, b2):
    return pl.pallas_call(
        _mlp_body,
        out_shape=jax.ShapeDtypeStruct((G, 1), jnp.float32),
    )(ps, gf, w1a, w1b, b1, w2, b2)


def _make_sc_edge(ct, dn, base0, GB):
    cl = ct // 16
    clh = cl // H
    mesh = plsc.VectorSubcoreMesh(core_axis_name="c", subcore_axis_name="s")

    def body(xl_hbm, xr_hbm, src_hbm, dst_hbm, att_hbm, out_hbm,
             sbuf, dbuf, attv, sstage, dstage, shead, dhead,
             rowsl, rowsr, md, outv, sem, sem2, sem3, sem4):
        wid = lax.axis_index("s") * 2 + lax.axis_index("c")
        base = base0 + wid * dn
        iota = lax.iota(jnp.int32, 16)
        f0 = jnp.zeros((16,), jnp.float32)

        pltpu.sync_copy(att_hbm, attv)

        def init_md(r, c):
            plsc.store_scatter(md, [r * 8 + (iota & 7)],
                               jnp.where(iota < 4, -3.0e38, 0.0),
                               mask=iota < 8)
            return c
        lax.fori_loop(0, dn, init_md, 0)

        def init_out(r, c):
            def init_j(j, c2):
                outv[r, pl.ds(j * 16, 16)] = f0
                return c2
            lax.fori_loop(0, cl, init_j, 0)
            return c
        lax.fori_loop(0, dn, init_out, 0)

        def init_stage(k, c):
            plsc.store_scatter(sstage, [k * 16 + iota],
                               jnp.zeros((16,), jnp.int32))
            plsc.store_scatter(dstage, [k * 16 + iota],
                               jnp.zeros((16,), jnp.int32))
            return c
        lax.fori_loop(0, STG // 16, init_stage, 0)

        def process_batch(boff, pcount):
            for t in range(GB // 16):
                sh = plsc.load_gather(sstage, [boff + t * 16 + iota])
                dh = plsc.load_gather(dstage, [boff + t * 16 + iota])
                shead[pl.ds(t * 16, 16)] = sh
                dhead[pl.ds(t * 16, 16)] = dh
            cpa = pltpu.async_copy(xl_hbm.at[shead], rowsl, sem)
            cpb = pltpu.async_copy(xr_hbm.at[dhead], rowsr, sem2)
            cpa.wait()
            cpb.wait()

            def edge_body(i, c):
                ifull = jnp.full((16,), i, jnp.int32)
                dspl = plsc.load_gather(dhead, [ifull])
                drel = dspl - base
                dsc = jnp.max(drel)
                st = md[pl.ds(dsc * 8, 16)]

                rowregs = []
                l4 = f0
                for h in range(H):
                    a1 = f0
                    a2 = f0
                    for j in range(clh):
                        jj = h * clh + j
                        col = jj * 16 + iota
                        vl = rowsl[i, pl.ds(jj * 16, 16)]
                        vr = rowsr[i, pl.ds(jj * 16, 16)]
                        av = attv[pl.ds(jj * 16, 16)]
                        v = vl + vr
                        a1 = a1 + av * v
                        a2 = a2 + av * jnp.maximum(v, 0.0)
                        rowregs.append(vl)
                    lh = jnp.sum(0.2 * a1 + 0.8 * a2)
                    l4 = jnp.where(iota == h, lh, l4)

                den_al = st.at[jnp.minimum(iota + 4, 15)].get(
                    mode="promise_in_bounds")
                nm = jnp.maximum(st, l4)
                scl = jnp.exp(st - nm)
                w = jnp.exp(l4 - nm)
                den_new = den_al * scl + w
                den_sh = den_new.at[jnp.maximum(iota - 4, 0)].get(
                    mode="promise_in_bounds")
                stn = jnp.where(iota < 4, nm, den_sh)
                plsc.store_scatter(md, [dsc * 8 + (iota & 7)], stn,
                                   mask=iota < 8)

                for h in range(H):
                    wh = w.at[jnp.full((16,), h, jnp.int32)].get(
                        mode="promise_in_bounds")
                    sh_ = scl.at[jnp.full((16,), h, jnp.int32)].get(
                        mode="promise_in_bounds")
                    for j in range(clh):
                        jj = h * clh + j
                        col = jj * 16 + iota
                        acc = outv[dsc, pl.ds(jj * 16, 16)]
                        acc = acc * sh_ + wh * rowregs[jj]
                        outv[dsc, pl.ds(jj * 16, 16)] = acc
                return c
            lax.fori_loop(0, pcount, edge_body, 0)

        def fire_chunk(ci):
            off = (ci % 2) * CHUNK
            pltpu.make_async_copy(src_hbm.at[pl.ds(ci * CHUNK, CHUNK)],
                                  sbuf.at[pl.ds(off, CHUNK)], sem3).start()
            pltpu.make_async_copy(dst_hbm.at[pl.ds(ci * CHUNK, CHUNK)],
                                  dbuf.at[pl.ds(off, CHUNK)], sem4).start()

        fire_chunk(0)

        def chunk_body(ci, cnt):
            off = (ci % 2) * CHUNK
            pltpu.make_async_copy(src_hbm.at[pl.ds(ci * CHUNK, CHUNK)],
                                  sbuf.at[pl.ds(off, CHUNK)], sem3).wait()
            pltpu.make_async_copy(dst_hbm.at[pl.ds(ci * CHUNK, CHUNK)],
                                  dbuf.at[pl.ds(off, CHUNK)], sem4).wait()

            @pl.when(ci + 1 < NCH)
            def _():
                fire_chunk(ci + 1)

            def grp_body(k, c):
                s = plsc.load_gather(sbuf, [off + k * 16 + iota])
                d = plsc.load_gather(dbuf, [off + k * 16 + iota])
                drel = d - base
                hit = (drel >= 0) & (drel < dn)
                hi = hit.astype(jnp.int32)
                nh = jnp.sum(hi)
                pos = jnp.maximum(c + plsc.cumsum(hi) - 1, 0)
                plsc.store_scatter(sstage, [pos], s, mask=hit)
                plsc.store_scatter(dstage, [pos], d, mask=hit)
                return c + nh
            cnt = lax.fori_loop(0, GRP, grp_body, cnt)

            nb = cnt // GB

            def batch_body(b, c):
                process_batch(b * GB, GB)
                return c
            lax.fori_loop(0, nb, batch_body, 0)

            rem = cnt - nb * GB
            roff = nb * GB
            for t in range(GB // 16):
                rs = plsc.load_gather(sstage, [roff + t * 16 + iota])
                rd = plsc.load_gather(dstage, [roff + t * 16 + iota])
                mk = (t * 16 + iota) < rem
                plsc.store_scatter(sstage, [t * 16 + iota], rs, mask=mk)
                plsc.store_scatter(dstage, [t * 16 + iota], rd, mask=mk)
            return rem
        cnt = lax.fori_loop(0, NCH, chunk_body, 0)

        @pl.when(cnt > 0)
        def _():
            process_batch(0, cnt)

        def norm_body(r, c):
            st = md[pl.ds(r * 8, 16)]
            for h in range(H):
                dh = st.at[jnp.full((16,), 4 + h, jnp.int32)].get(
                    mode="promise_in_bounds")
                inv = 1.0 / (dh + 1e-16)
                for j in range(clh):
                    jj = h * clh + j
                    v = outv[r, pl.ds(jj * 16, 16)]
                    outv[r, pl.ds(jj * 16, 16)] = v * inv
            return c
        lax.fori_loop(0, dn, norm_body, 0)

        pltpu.sync_copy(outv, out_hbm.at[pl.ds(wid * dn, dn)])

    fn = functools.partial(
        pl.kernel, body,
        mesh=mesh,
        compiler_params=pltpu.CompilerParams(use_tc_tiling_on_sc=False,
                                             needs_layout_passes=False),
        out_type=jax.ShapeDtypeStruct((NW * dn, ct), jnp.float32),
        scratch_types=[
            pltpu.VMEM((2 * CHUNK,), jnp.int32),
            pltpu.VMEM((2 * CHUNK,), jnp.int32),
            pltpu.VMEM((ct,), jnp.float32),
            pltpu.VMEM((STG,), jnp.int32),
            pltpu.VMEM((STG,), jnp.int32),
            pltpu.VMEM((GB,), jnp.int32),
            pltpu.VMEM((GB,), jnp.int32),
            pltpu.VMEM((GB, ct), jnp.float32),
            pltpu.VMEM((GB, ct), jnp.float32),
            pltpu.VMEM((dn * 8 + 8,), jnp.float32),
            pltpu.VMEM((dn, ct), jnp.float32),
            pltpu.SemaphoreType.DMA,
            pltpu.SemaphoreType.DMA,
            pltpu.SemaphoreType.DMA,
            pltpu.SemaphoreType.DMA,
        ],
    )
    return fn()


def _gat_layer_sc(xl, xr, src, dst, att, splits):
    ct = xl.shape[1]
    att_flat = att.reshape(ct)
    outs = []
    gb = 64 if ct <= 256 else 32
    for dn, base0 in splits:
        sc = _make_sc_edge(ct, dn, base0, gb)
        outs.append(sc(xl, xr, src, dst, att_flat))
    o = outs[0] if len(outs) == 1 else jnp.concatenate(outs, axis=0)
    return o[:N]


def kernel(x, edge_index, batch, global_feat, W1l, W1r, att1, b1, g1, be1,
           W2l, W2r, att2, b2, g2, be2, fcW1, fcb1, fcW2, fcb2):
    src = edge_index[0]
    dst = edge_index[1]

    xl1, xr1 = _tc_two_matmuls(x, W1l, W1r)
    o1 = _gat_layer_sc(xl1, xr1, src, dst, att1, [(320, 0)])
    s1, q1 = _tc_stats(o1, b1.reshape(1, -1))
    xl2, xr2 = _tc_mid(o1, b1.reshape(1, -1), s1, q1, g1.reshape(1, -1),
                       be1.reshape(1, -1), W2l, W2r)

    o2 = _gat_layer_sc(xl2, xr2, src, dst, att2, [(160, 0), (160, 5120)])
    s2, q2 = _tc_stats(o2, b2.reshape(1, -1))

    batch3 = batch.reshape(NB, 1, BR)
    ps = _tc_post(o2, b2.reshape(1, -1), s2, q2, g2.reshape(1, -1),
                  be2.reshape(1, -1), batch3)

    out = _tc_mlp(ps, global_feat, fcW1[:512], fcW1[512:],
                  fcb1.reshape(1, -1), fcW2, fcb2.reshape(1, 1))
    return out.squeeze()
